# baseline (speedup 1.0000x reference)
import jax
import jax.numpy as jnp
from jax.experimental import pallas as pl
from jax.experimental.pallas import tpu as pltpu
from jax.experimental.pallas import tpu_sc as plsc

_D = 512
_PAIRS = 64


def _body(l1_hbm, l0_hbm, out_hbm, s1, sp):
    c1 = pltpu.make_async_copy(
        l1_hbm.at[pl.ds(0, 128)], out_hbm.at[pl.ds(0, 128)], s1
    )
    c1.start()
    pairs = []
    for k in range(_PAIRS):
        c = pltpu.make_async_copy(
            l0_hbm.at[pl.ds(4 * k, 2)],
            out_hbm.at[pl.ds(128 + 2 * k, 2)],
            sp,
        )
        pairs.append(c)
    for c in pairs:
        c.start()
    c1.wait()
    for c in pairs:
        c.wait()


@jax.jit
def kernel(layer1, layer0):
    mesh = plsc.ScalarSubcoreMesh(axis_name="c", num_cores=1)
    f = pl.kernel(
        _body,
        out_type=jax.ShapeDtypeStruct((256, _D), jnp.float32),
        mesh=mesh,
        scratch_types=[
            pltpu.SemaphoreType.DMA,
            pltpu.SemaphoreType.DMA,
        ],
    )
    return f(layer1, layer0)


# speedup vs baseline: 1.3224x; 1.3224x over previous
import jax
import jax.numpy as jnp
from jax.experimental import pallas as pl
from jax.experimental.pallas import tpu as pltpu
from jax.experimental.pallas import tpu_sc as plsc

_D = 512
_GROUPS = 4
_PAIRS_PER_GROUP = 16


def _body(l1_hbm, l0_hbm, out_hbm, buf, s1, sg, so):
    c1 = pltpu.make_async_copy(
        l1_hbm.at[pl.ds(0, 128)], out_hbm.at[pl.ds(0, 128)], s1
    )
    c1.start()
    groups = []
    for g in range(_GROUPS):
        grp = []
        for j in range(_PAIRS_PER_GROUP):
            k = g * _PAIRS_PER_GROUP + j
            grp.append(
                pltpu.make_async_copy(
                    l0_hbm.at[pl.ds(4 * k, 2)],
                    buf.at[pl.ds(2 * k, 2)],
                    sg[g],
                )
            )
        groups.append(grp)
    for grp in groups:
        for c in grp:
            c.start()

    outs = []
    for g, grp in enumerate(groups):
        for c in grp:
            c.wait()
        base = g * 2 * _PAIRS_PER_GROUP
        o = pltpu.make_async_copy(
            buf.at[pl.ds(base, 2 * _PAIRS_PER_GROUP)],
            out_hbm.at[pl.ds(128 + base, 2 * _PAIRS_PER_GROUP)],
            so,
        )
        o.start()
        outs.append(o)
    c1.wait()
    for o in outs:
        o.wait()


@jax.jit
def kernel(layer1, layer0):
    mesh = plsc.ScalarSubcoreMesh(axis_name="c", num_cores=1)
    f = pl.kernel(
        _body,
        out_type=jax.ShapeDtypeStruct((256, _D), jnp.float32),
        mesh=mesh,
        scratch_types=[
            pltpu.VMEM_SHARED((128, _D), jnp.float32),
            pltpu.SemaphoreType.DMA,
            [pltpu.SemaphoreType.DMA] * _GROUPS,
            pltpu.SemaphoreType.DMA,
        ],
    )
    return f(layer1, layer0)


# speedup vs baseline: 1.8023x; 1.3629x over previous
import jax
import jax.numpy as jnp
from jax.experimental import pallas as pl
from jax.experimental.pallas import tpu as pltpu
from jax.experimental.pallas import tpu_sc as plsc

_D = 512
_GROUPS = 4
_PAIRS_PER_GROUP = 16


def _body(l1_hbm, l0_hbm, out_hbm, buf, s1a, s1b, sg, so):
    c1a = pltpu.make_async_copy(l1_hbm.at[pl.ds(0, 64)], buf.at[pl.ds(0, 64)], s1a)
    c1b = pltpu.make_async_copy(l1_hbm.at[pl.ds(64, 64)], buf.at[pl.ds(64, 64)], s1b)
    c1a.start()
    c1b.start()
    groups = []
    for g in range(_GROUPS):
        grp = []
        for j in range(_PAIRS_PER_GROUP):
            k = g * _PAIRS_PER_GROUP + j
            grp.append(
                pltpu.make_async_copy(
                    l0_hbm.at[pl.ds(4 * k, 2)],
                    buf.at[pl.ds(128 + 2 * k, 2)],
                    sg[g],
                )
            )
        groups.append(grp)
    for grp in groups:
        for c in grp:
            c.start()

    outs = []
    c1a.wait()
    o = pltpu.make_async_copy(buf.at[pl.ds(0, 64)], out_hbm.at[pl.ds(0, 64)], so)
    o.start()
    outs.append(o)
    c1b.wait()
    o = pltpu.make_async_copy(buf.at[pl.ds(64, 64)], out_hbm.at[pl.ds(64, 64)], so)
    o.start()
    outs.append(o)
    for g, grp in enumerate(groups):
        for c in grp:
            c.wait()
        base = 128 + g * 2 * _PAIRS_PER_GROUP
        o = pltpu.make_async_copy(
            buf.at[pl.ds(base, 2 * _PAIRS_PER_GROUP)],
            out_hbm.at[pl.ds(base, 2 * _PAIRS_PER_GROUP)],
            so,
        )
        o.start()
        outs.append(o)
    for o in outs:
        o.wait()


@jax.jit
def kernel(layer1, layer0):
    mesh = plsc.ScalarSubcoreMesh(axis_name="c", num_cores=1)
    f = pl.kernel(
        _body,
        out_type=jax.ShapeDtypeStruct((256, _D), jnp.float32),
        mesh=mesh,
        scratch_types=[
            pltpu.VMEM_SHARED((256, _D), jnp.float32),
            pltpu.SemaphoreType.DMA,
            pltpu.SemaphoreType.DMA,
            [pltpu.SemaphoreType.DMA] * _GROUPS,
            pltpu.SemaphoreType.DMA,
        ],
    )
    return f(layer1, layer0)
